# Initial kernel scaffold; baseline (speedup 1.0000x reference)
import jax
import jax.numpy as jnp
from jax.experimental import pallas as pl


def kernel(x, edge_index, batch, W1, b1, W2, b2, Wseq, bseq, Wlin, blin):
    raise NotImplementedError("write your pallas kernel here")


# speedup vs baseline: 11.0070x; 11.0070x over previous
import functools
import jax
import jax.numpy as jnp
from jax import lax
from jax.experimental import pallas as pl
from jax.experimental.pallas import tpu as pltpu
from jax.experimental.pallas import tpu_sc as plsc

N = 10000
NPAD = 10112
STRIPE = NPAD // 16
E = 320000
NW = 32
SHOTS = 125
B = 80
BN = 1000
NBLK = N // BN
F = 128
H = 256
G = 128
C = 16

_mesh = plsc.VectorSubcoreMesh(core_axis_name="c", subcore_axis_name="s")


@functools.partial(
    pl.kernel,
    out_type=jax.ShapeDtypeStruct((4, 2, NPAD, F), jnp.float32),
    mesh=_mesh,
    scratch_types=[
        pltpu.VMEM((16,), jnp.int32),
        pltpu.VMEM((SHOTS, B), jnp.int32),
        pltpu.VMEM((SHOTS, B), jnp.int32),
        pltpu.VMEM((B, F), jnp.float32),
        pltpu.VMEM_SHARED((NPAD, F), jnp.float32),
        pltpu.SemaphoreType.DMA,
    ],
)
def _prop_kernel(h_hbm, src_hbm, dst_hbm, nch_hbm, out_hbm,
                 nch_v, src_v, dst_v, rows_v, acc_sh, sem):
    c = lax.axis_index("c")
    s = lax.axis_index("s")
    w = c * 16 + s
    base = s * STRIPE

    pltpu.sync_copy(nch_hbm, nch_v)
    pltpu.sync_copy(src_hbm.at[w], src_v)
    pltpu.sync_copy(dst_hbm.at[w], dst_v)
    nch = nch_v[pl.ds(0, 16)][0]

    def _fill(val):
        def _row(r, carry):
            for q in range(F // 16):
                rows_v[r, pl.ds(q * 16, 16)] = jnp.full((16,), val,
                                                        jnp.float32)
            return carry
        lax.fori_loop(0, B, _row, 0)

    def _sweep(cc, gather):
        _fill(0.0)
        for k in range(8):
            pltpu.sync_copy(rows_v.at[pl.ds(0, STRIPE // 8)],
                            acc_sh.at[pl.ds(base + k * (STRIPE // 8),
                                            STRIPE // 8)])
        if not gather:
            _fill(1.0)
        plsc.subcore_barrier()

        def _gstep(j, inner):
            pltpu.async_copy(h_hbm.at[cc].at[src_v.at[j]], rows_v, sem).wait()
            pltpu.sync_copy(rows_v, acc_sh.at[dst_v.at[j]], add=True)
            return inner

        def _ostep(j, inner):
            pltpu.sync_copy(rows_v, acc_sh.at[dst_v.at[j]], add=True)
            return inner

        lax.fori_loop(0, SHOTS, _gstep if gather else _ostep, 0)
        plsc.subcore_barrier()
        pltpu.sync_copy(acc_sh.at[pl.ds(base, STRIPE)],
                        out_hbm.at[cc, c, pl.ds(base, STRIPE)])

    def _deg_mode():
        _sweep(0, gather=False)

    def _prop_mode():
        def _body(cc, car):
            lax.cond(cc < nch, lambda: _sweep(cc, gather=True), lambda: None)
            return car
        lax.fori_loop(0, 4, _body, 0)

    lax.cond(nch == 0, _deg_mode, _prop_mode)


def _dinv_of(dp_ref):
    return lax.rsqrt(1.0 + dp_ref[0, 0, :, 0] + dp_ref[0, 1, :, 0])


def _tc0_body(x_ref, dp_ref, xs_ref):
    dinv = _dinv_of(dp_ref)
    xs_ref[0] = x_ref[...] * dinv[:, None]


def _tc0(x, dp):
    return pl.pallas_call(
        _tc0_body,
        grid=(NBLK,),
        in_specs=[
            pl.BlockSpec((BN, F), lambda i: (i, 0)),
            pl.BlockSpec((1, 2, BN, F), lambda i: (0, 0, i, 0)),
        ],
        out_specs=pl.BlockSpec((4, BN, F), lambda i: (0, i, 0)),
        out_shape=jax.ShapeDtypeStruct((4, N, F), jnp.float32),
    )(x, dp)


def _sum_parts(p_ref, h_ref, lo, hi):
    return jnp.concatenate(
        [p_ref[cc, 0] + p_ref[cc, 1] + h_ref[cc] for cc in range(lo, hi)],
        axis=1)


def _tc1_body(zp_ref, xs_ref, dp_ref, w1_ref, b1_ref, w2_ref, b2_ref, out_ref):
    dinv = _dinv_of(dp_ref)
    z = zp_ref[0, 0] + zp_ref[0, 1] + xs_ref[0]
    t = z * dinv[:, None]
    a1 = jnp.maximum(jnp.dot(t, w1_ref[...],
                             preferred_element_type=jnp.float32)
                     + b1_ref[...], 0.0) * dinv[:, None]
    a2 = jnp.maximum(jnp.dot(t, w2_ref[...],
                             preferred_element_type=jnp.float32)
                     + b2_ref[...], 0.0) * dinv[:, None]
    out_ref[0] = a1[:, :F]
    out_ref[1] = a1[:, F:]
    out_ref[2] = a2[:, :F]
    out_ref[3] = a2[:, F:]


def _tc1(zp, xs, dp, W1, b1, W2, b2):
    return pl.pallas_call(
        _tc1_body,
        grid=(NBLK,),
        in_specs=[
            pl.BlockSpec((1, 2, BN, F), lambda i: (0, 0, i, 0)),
            pl.BlockSpec((1, BN, F), lambda i: (0, i, 0)),
            pl.BlockSpec((1, 2, BN, F), lambda i: (0, 0, i, 0)),
            pl.BlockSpec((F, H), lambda i: (0, 0)),
            pl.BlockSpec((1, H), lambda i: (0, 0)),
            pl.BlockSpec((F, H), lambda i: (0, 0)),
            pl.BlockSpec((1, H), lambda i: (0, 0)),
        ],
        out_specs=pl.BlockSpec((4, BN, F), lambda i: (0, i, 0)),
        out_shape=jax.ShapeDtypeStruct((4, N, F), jnp.float32),
    )(zp, xs, dp, W1, b1, W2, b2)


def _tc2_body(qp_ref, h4_ref, dp_ref, wa_ref, ba_ref, wb_ref, bb_ref, out_ref):
    dinv = _dinv_of(dp_ref)
    q1 = _sum_parts(qp_ref, h4_ref, 0, 2)
    q2 = _sum_parts(qp_ref, h4_ref, 2, 4)
    y1 = jnp.maximum(jnp.dot(q1 * dinv[:, None], wa_ref[...],
                             preferred_element_type=jnp.float32)
                     + ba_ref[...], 0.0) * dinv[:, None]
    y2 = jnp.maximum(jnp.dot(q2 * dinv[:, None], wb_ref[...],
                             preferred_element_type=jnp.float32)
                     + bb_ref[...], 0.0) * dinv[:, None]
    u1 = y1 + jnp.concatenate([h4_ref[2], h4_ref[3]], axis=1)
    u2 = y2 + jnp.concatenate([h4_ref[0], h4_ref[1]], axis=1)
    out_ref[0] = u1[:, :F]
    out_ref[1] = u1[:, F:]
    out_ref[2] = u2[:, :F]
    out_ref[3] = u2[:, F:]


def _tc2(qp, h4, dp, Wa, ba, Wb, bb):
    return pl.pallas_call(
        _tc2_body,
        grid=(NBLK,),
        in_specs=[
            pl.BlockSpec((4, 2, BN, F), lambda i: (0, 0, i, 0)),
            pl.BlockSpec((4, BN, F), lambda i: (0, i, 0)),
            pl.BlockSpec((1, 2, BN, F), lambda i: (0, 0, i, 0)),
            pl.BlockSpec((H, H), lambda i: (0, 0)),
            pl.BlockSpec((1, H), lambda i: (0, 0)),
            pl.BlockSpec((H, H), lambda i: (0, 0)),
            pl.BlockSpec((1, H), lambda i: (0, 0)),
        ],
        out_specs=pl.BlockSpec((4, BN, F), lambda i: (0, i, 0)),
        out_shape=jax.ShapeDtypeStruct((4, N, F), jnp.float32),
    )(qp, h4, dp, Wa, ba, Wb, bb)


def _tc3_body(rp_ref, u4_ref, dp_ref, wa_ref, ba_ref, wb_ref, bb_ref,
              batch_ref, wlin_ref, blin_ref, y_ref, gm_ref, acc_ref, cnt_ref):
    i = pl.program_id(0)
    dinv = _dinv_of(dp_ref)
    r1 = _sum_parts(rp_ref, u4_ref, 0, 2)
    r2 = _sum_parts(rp_ref, u4_ref, 2, 4)
    c1 = jnp.maximum(jnp.dot(r1 * dinv[:, None], wa_ref[...],
                             preferred_element_type=jnp.float32)
                     + ba_ref[...], 0.0)
    c2 = jnp.maximum(jnp.dot(r2 * dinv[:, None], wb_ref[...],
                             preferred_element_type=jnp.float32)
                     + bb_ref[...], 0.0)
    xc = c1 + c2

    bb_ids = batch_ref[0, 0, :]
    gids = lax.broadcasted_iota(jnp.int32, (G, BN), 0)
    m = (gids == bb_ids[None, :]).astype(jnp.float32)
    part = jnp.dot(m, xc, preferred_element_type=jnp.float32)
    pc = jnp.sum(m, axis=1, keepdims=True)

    @pl.when(i == 0)
    def _init():
        acc_ref[...] = part
        cnt_ref[...] = pc

    @pl.when(i > 0)
    def _accum():
        acc_ref[...] += part
        cnt_ref[...] += pc

    @pl.when(i == NBLK - 1)
    def _final():
        gm = acc_ref[...] / jnp.maximum(cnt_ref[...], 1.0)
        gm_ref[...] = gm
        y_ref[...] = jnp.dot(gm, wlin_ref[...],
                             preferred_element_type=jnp.float32) + blin_ref[...]


def _tc3(rp, u4, dp, Wa, ba, Wb, bb, batch3, Wlin, blin):
    return pl.pallas_call(
        _tc3_body,
        grid=(NBLK,),
        in_specs=[
            pl.BlockSpec((4, 2, BN, F), lambda i: (0, 0, i, 0)),
            pl.BlockSpec((4, BN, F), lambda i: (0, i, 0)),
            pl.BlockSpec((1, 2, BN, F), lambda i: (0, 0, i, 0)),
            pl.BlockSpec((H, H), lambda i: (0, 0)),
            pl.BlockSpec((1, H), lambda i: (0, 0)),
            pl.BlockSpec((H, H), lambda i: (0, 0)),
            pl.BlockSpec((1, H), lambda i: (0, 0)),
            pl.BlockSpec((1, 1, BN), lambda i: (i, 0, 0)),
            pl.BlockSpec((H, C), lambda i: (0, 0)),
            pl.BlockSpec((1, C), lambda i: (0, 0)),
        ],
        out_specs=[
            pl.BlockSpec((G, C), lambda i: (0, 0)),
            pl.BlockSpec((G, H), lambda i: (0, 0)),
        ],
        out_shape=[
            jax.ShapeDtypeStruct((G, C), jnp.float32),
            jax.ShapeDtypeStruct((G, H), jnp.float32),
        ],
        scratch_shapes=[
            pltpu.VMEM((G, H), jnp.float32),
            pltpu.VMEM((G, 1), jnp.float32),
        ],
    )(rp, u4, dp, Wa, ba, Wb, bb, batch3, Wlin, blin)


def kernel(x, edge_index, batch, W1, b1, W2, b2, Wseq, bseq, Wlin, blin):
    src3 = edge_index[0].reshape(NW, SHOTS, B)
    dst3 = edge_index[1].reshape(NW, SHOTS, B)
    batch3 = batch.reshape(NBLK, 1, BN)
    b1r = b1.reshape(1, H)
    b2r = b2.reshape(1, H)
    bs = [bseq[i].reshape(1, H) for i in range(4)]
    blinr = blin.reshape(1, C)
    nch_rounds = jnp.array([0, 1, 4, 4], jnp.int32)[:, None] * jnp.ones(
        (1, 16), jnp.int32)
    h0 = jnp.zeros((4, N, F), jnp.float32)
    dp0 = jnp.zeros((4, 2, NPAD, F), jnp.float32)

    def _round(carry, xr):
        h, dp, _ = carry
        r, nch = xr
        parts = _prop_kernel(h, src3, dst3, nch)

        def b0():
            return _tc0(x, parts), parts

        def b1():
            return _tc1(parts, h, dp, W1, b1r, W2, b2r), dp

        def b2():
            return _tc2(parts, h, dp, Wseq[0], bs[0], Wseq[1], bs[1]), dp

        def b3():
            return h, dp

        h2, dp2 = lax.switch(r, [b0, b1, b2, b3])
        return (h2, dp2, parts), None

    (u4, dp, rp), _ = lax.scan(
        _round, (h0, dp0, dp0), (jnp.arange(4), nch_rounds))
    y, gm = _tc3(rp, u4, dp, Wseq[2], bs[2], Wseq[3], bs[3],
                 batch3, Wlin, blinr)
    return (y, gm)
